# baseline (speedup 1.0000x reference)
import functools

import jax
import jax.numpy as jnp
from jax import lax
from jax.experimental import pallas as pl
from jax.experimental.pallas import tpu as pltpu
from jax.experimental.pallas import tpu_sc as plsc

MIN_NORM = 1e-15
BALL_EPS = 0.004
MIN_CURVATURE = 1e-06

N_NODE = 10000
N_EDGE = 320000
D = 128
DP = 64
NPAD = 10112
NBLK = 79
NCORE = 2
NSUB = 16
NW = NCORE * NSUB
EPW = N_EDGE // NW
CH = 80
NCH = EPW // CH
RPT = N_NODE // NSUB
EB = 512
NEB = N_EDGE // EB


def _tanh_c(x):
    return jnp.tanh(jnp.clip(x, -15.0, 15.0))


def _artanh(x):
    x = jnp.clip(x, -1.0 + 1e-05, 1.0 - 1e-05)
    return 0.5 * (jnp.log1p(x) - jnp.log1p(-x))


def _expmap0(u, c):
    sqrt_c = jnp.sqrt(c)
    u_norm = jnp.maximum(jnp.sqrt(jnp.sum(u * u, axis=-1, keepdims=True)), MIN_NORM)
    gamma = _tanh_c(sqrt_c * u_norm) * u / (sqrt_c * u_norm)
    norm = jnp.maximum(jnp.sqrt(jnp.sum(gamma * gamma, axis=-1, keepdims=True)), MIN_NORM)
    maxnorm = (1.0 - BALL_EPS) / sqrt_c
    return jnp.where(norm > maxnorm, gamma / norm * maxnorm, gamma)


def _pre_body(c_ref, hid, rela, path, Ws, Wr, Wq, bq, Wpp, Wpr,
              AS, AR, AQ, PS, PR, HS, HR):
    c = jnp.maximum(c_ref[0, 0], MIN_CURVATURE)
    h = hid[...]
    r = rela[...]
    AS[...] = jax.lax.dot_general(h, Ws[...], (((1,), (0,)), ((), ())),
                                  preferred_element_type=jnp.float32)
    AR[...] = jax.lax.dot_general(r, Wr[...], (((1,), (0,)), ((), ())),
                                  preferred_element_type=jnp.float32)
    AQ[...] = jax.lax.dot_general(r, Wq[...], (((1,), (0,)), ((), ())),
                                  preferred_element_type=jnp.float32) + bq[...]
    PS[...] = jax.lax.dot_general(path[...], Wpp[...], (((1,), (0,)), ((), ())),
                                  preferred_element_type=jnp.float32)
    PR[...] = jax.lax.dot_general(r, Wpr[...], (((1,), (0,)), ((), ())),
                                  preferred_element_type=jnp.float32)
    HS[...] = _expmap0(h, c)
    HR[...] = _expmap0(r, c)


def _precompute(cc, hid_p, rela_p, path_p, Ws, Wr, Wq, bq, Wpp, Wpr):
    f32 = jnp.float32
    full = lambda s: pl.BlockSpec(s, lambda i: (0,) * len(s))
    row128 = pl.BlockSpec((D, D), lambda i: (i, 0))
    row64 = pl.BlockSpec((D, DP), lambda i: (i, 0))
    return pl.pallas_call(
        _pre_body,
        grid=(NBLK,),
        in_specs=[
            pl.BlockSpec(memory_space=pltpu.SMEM),
            row128, row128, row64,
            full((D, D)), full((D, D)), full((D, D)), full((1, D)),
            full((DP, DP)), full((D, DP)),
        ],
        out_specs=[row128, row128, row128, row64, row64, row128, row128],
        out_shape=[
            jax.ShapeDtypeStruct((NPAD, D), f32),
            jax.ShapeDtypeStruct((NPAD, D), f32),
            jax.ShapeDtypeStruct((NPAD, D), f32),
            jax.ShapeDtypeStruct((NPAD, DP), f32),
            jax.ShapeDtypeStruct((NPAD, DP), f32),
            jax.ShapeDtypeStruct((NPAD, D), f32),
            jax.ShapeDtypeStruct((NPAD, D), f32),
        ],
    )(cc, hid_p, rela_p, path_p, Ws, Wr, Wq, bq, Wpp, Wpr)


_LN2 = 0.6931471805599453


def _v_rsqrt(x):
    i = plsc.bitcast(x, jnp.int32)
    y = plsc.bitcast(jnp.int32(0x5F3759DF) - (i >> 1), jnp.float32)
    for _ in range(3):
        y = y * (1.5 - 0.5 * x * y * y)
    return y


def _v_ln(u):
    bits = plsc.bitcast(u, jnp.int32)
    ex = jnp.astype((bits >> 23) - 127, jnp.float32)
    m = plsc.bitcast((bits & jnp.int32(0x007FFFFF)) | jnp.int32(0x3F800000),
                     jnp.float32)
    t = (m - 1.0) / (m + 1.0)
    t2 = t * t
    s = 1.0 / 9.0 + t2 * (1.0 / 11.0)
    s = 1.0 / 7.0 + t2 * s
    s = 1.0 / 5.0 + t2 * s
    s = 1.0 / 3.0 + t2 * s
    s = 1.0 + t2 * s
    return ex * _LN2 + 2.0 * t * s


def _fused_body(qrel_hbm, ridx_hbm, sub_hbm, rel_hbm, obj_hbm, wa_hbm, wb_hbm,
                cvec_hbm, z_hbm,
                AS_hbm, AR_hbm, AQ_hbm, HS_hbm, HR_hbm, PS_hbm, PR_hbm,
                MSG, ppart,
                ridx_v, sub_v, rel_v, qi_v, obj_v, wa_v, sc_v,
                b_as, b_ar, b_aq, b_hs, b_hr, b_ps, b_pr,
                pacc, sem, sem2):
    cid = lax.axis_index("c")
    sid = lax.axis_index("s")
    wslot = cid * NSUB + sid
    base = wslot * EPW
    lanes = lax.iota(jnp.int32, 16)

    pltpu.sync_copy(z_hbm.at[pl.ds(sid * RPT, RPT)], pacc.at[pl.ds(sid * RPT, RPT)])
    pltpu.sync_copy(wa_hbm, wa_v)
    pltpu.sync_copy(cvec_hbm, sc_v)
    plsc.subcore_barrier()

    cv = jnp.maximum(sc_v[...], MIN_CURVATURE)
    sqrt_c = cv * _v_rsqrt(cv)
    maxnorm = (1.0 - BALL_EPS) / sqrt_c
    pltpu.async_copy(wb_hbm, sc_v, sem).wait()
    wbv = sc_v[...]

    def chunk(k, _):
        off = base + k * CH
        pltpu.sync_copy(ridx_hbm.at[pl.ds(off, CH)], ridx_v)
        pltpu.sync_copy(sub_hbm.at[pl.ds(off, CH)], sub_v)
        pltpu.sync_copy(rel_hbm.at[pl.ds(off, CH)], rel_v)
        pltpu.sync_copy(obj_hbm.at[wslot, k], obj_v)
        pltpu.async_copy(qrel_hbm.at[ridx_v], qi_v, sem).wait()
        cs = [
            pltpu.async_copy(AS_hbm.at[sub_v], b_as, sem),
            pltpu.async_copy(AR_hbm.at[rel_v], b_ar, sem),
            pltpu.async_copy(AQ_hbm.at[qi_v], b_aq, sem),
            pltpu.async_copy(HS_hbm.at[sub_v], b_hs, sem),
            pltpu.async_copy(HR_hbm.at[rel_v], b_hr, sem),
            pltpu.async_copy(PS_hbm.at[sub_v], b_ps, sem),
            pltpu.async_copy(PR_hbm.at[rel_v], b_pr, sem),
        ]
        for h in cs:
            h.wait()

        def group(g, _):
            eidx = lanes + g * 16

            def dots(j, carry):
                logitv, xyv, x2v, y2v = carry
                jf = jnp.full((16,), j, jnp.int32)
                av = (plsc.load_gather(b_as, [eidx, jf])
                      + plsc.load_gather(b_ar, [eidx, jf])
                      + plsc.load_gather(b_aq, [eidx, jf]))
                waj = plsc.load_gather(wa_v, [jf])
                logitv = logitv + jnp.maximum(av, 0.0) * waj
                hsj = plsc.load_gather(b_hs, [eidx, jf])
                hrj = plsc.load_gather(b_hr, [eidx, jf])
                xyv = xyv + hsj * hrj
                x2v = x2v + hsj * hsj
                y2v = y2v + hrj * hrj
                return logitv, xyv, x2v, y2v

            zero = jnp.zeros((16,), jnp.float32)
            logitv, xyv, x2v, y2v = lax.fori_loop(
                0, D, dots, (zero, zero, zero, zero))

            lg = jnp.clip(logitv + wbv, -30.0, 30.0)
            alpha = 1.0 / (1.0 + jnp.exp(-lg))
            A = 1.0 + 2.0 * cv * xyv + cv * y2v
            B = 1.0 - cv * x2v
            den = jnp.maximum(1.0 + 2.0 * cv * xyv + cv * cv * x2v * y2v,
                              MIN_NORM)
            r2 = jnp.maximum(A * A * x2v + 2.0 * A * B * xyv + B * B * y2v, 0.0)
            nm0 = jnp.maximum(r2 * _v_rsqrt(r2) / den, MIN_NORM)
            s1 = jnp.where(nm0 > maxnorm, maxnorm / nm0, 1.0) / den
            n2 = jnp.maximum(jnp.minimum(nm0, maxnorm), MIN_NORM)
            z = jnp.minimum(sqrt_c * n2, 1.0 - 1e-05)
            u = (1.0 + z) / (1.0 - z)
            factor = 0.5 * _v_ln(u) / z
            pv = alpha * factor * s1 * A
            qv = alpha * factor * s1 * B

            def emit_msg(j, _):
                jf = jnp.full((16,), j, jnp.int32)
                hsj = plsc.load_gather(b_hs, [eidx, jf])
                hrj = plsc.load_gather(b_hr, [eidx, jf])
                plsc.store_scatter(b_aq, [eidx, jf], hsj * pv + hrj * qv)
                return 0

            lax.fori_loop(0, D, emit_msg, 0)

            def emit_pe(j, _):
                jf = jnp.full((16,), j, jnp.int32)
                x = (plsc.load_gather(b_ps, [eidx, jf])
                     + plsc.load_gather(b_pr, [eidx, jf]))
                x = jnp.clip(x, -15.0, 15.0)
                ex = jnp.exp(2.0 * x)
                plsc.store_scatter(b_ps, [eidx, jf], alpha * ((ex - 1.0) / (ex + 1.0)))
                return 0

            lax.fori_loop(0, DP, emit_pe, 0)
            return 0

        lax.fori_loop(0, CH // 16, group, 0)
        pltpu.sync_copy(b_aq, MSG.at[pl.ds(off, CH)])
        pltpu.sync_copy(b_ps, pacc.at[obj_v.at[0]], add=True)
        return 0

    lax.fori_loop(0, NCH, chunk, 0)
    plsc.subcore_barrier()
    pltpu.sync_copy(pacc.at[pl.ds(sid * RPT, RPT)],
                    ppart.at[cid].at[pl.ds(sid * RPT, RPT)])


def _fused_edge(qrel, ridx, sub, rel, obj4, wa, wb16, cvec, z64,
                AS, AR, AQ, HS, HR, PS, PR):
    f32 = jnp.float32
    i32 = jnp.int32
    mesh = plsc.VectorSubcoreMesh(core_axis_name="c", subcore_axis_name="s",
                                  num_cores=NCORE, num_subcores=NSUB)
    out_type = (
        jax.ShapeDtypeStruct((N_EDGE, D), f32),
        jax.ShapeDtypeStruct((NCORE, N_NODE, DP), f32),
    )
    scratch = [
        pltpu.VMEM((CH,), i32),
        pltpu.VMEM((CH,), i32),
        pltpu.VMEM((CH,), i32),
        pltpu.VMEM((CH,), i32),
        pltpu.VMEM((1, CH), i32),
        pltpu.VMEM((D,), f32),
        pltpu.VMEM((16,), f32),
        pltpu.VMEM((CH, D), f32),
        pltpu.VMEM((CH, D), f32),
        pltpu.VMEM((CH, D), f32),
        pltpu.VMEM((CH, D), f32),
        pltpu.VMEM((CH, D), f32),
        pltpu.VMEM((CH, DP), f32),
        pltpu.VMEM((CH, DP), f32),
        pltpu.VMEM_SHARED((N_NODE, DP), f32),
        pltpu.SemaphoreType.DMA,
        pltpu.SemaphoreType.DMA,
    ]
    k = pl.kernel(_fused_body, out_type=out_type, mesh=mesh, scratch_types=scratch,
                  compiler_params=pltpu.CompilerParams(use_tc_tiling_on_sc=False,
                                                       needs_layout_passes=False))
    return k(qrel, ridx, sub, rel, obj4, wa, wb16, cvec, z64,
             AS, AR, AQ, HS, HR, PS, PR)


def _scatter_body(msg_hbm, obj_hbm, z_hbm, part,
                  obj_v, buf, acc):
    cid = lax.axis_index("c")
    sid = lax.axis_index("s")
    wslot = cid * NSUB + sid
    ebase = wslot * EPW
    pltpu.sync_copy(z_hbm.at[pl.ds(sid * RPT, RPT)], acc.at[pl.ds(sid * RPT, RPT)])
    pltpu.sync_copy(obj_hbm.at[wslot], obj_v)
    plsc.subcore_barrier()

    def step(k, _):
        off = k * CH
        pltpu.sync_copy(msg_hbm.at[pl.ds(ebase + off, CH)], buf)
        pltpu.sync_copy(buf, acc.at[obj_v.at[k]], add=True)
        return 0

    lax.fori_loop(0, NCH, step, 0)
    plsc.subcore_barrier()
    pltpu.sync_copy(acc.at[pl.ds(sid * RPT, RPT)],
                    part.at[cid].at[pl.ds(sid * RPT, RPT)])


def _scatter(msg, obj3, z, width):
    f32 = jnp.float32
    i32 = jnp.int32
    mesh = plsc.VectorSubcoreMesh(core_axis_name="c", subcore_axis_name="s",
                                  num_cores=NCORE, num_subcores=NSUB)
    out_type = jax.ShapeDtypeStruct((NCORE, N_NODE, width), f32)
    scratch = [
        pltpu.VMEM((NCH, CH), i32),
        pltpu.VMEM((CH, width), f32),
        pltpu.VMEM_SHARED((N_NODE, width), f32),
    ]
    k = pl.kernel(_scatter_body, out_type=out_type, mesh=mesh, scratch_types=scratch,
                  compiler_params=pltpu.CompilerParams(use_tc_tiling_on_sc=False))
    return k(msg, obj3, z)


def _final_body(c_ref, Wh, mp, pp, out1, out2):
    c = jnp.maximum(c_ref[0, 0], MIN_CURVATURE)
    sqrt_c = jnp.sqrt(c)
    magg = mp[0] + mp[1]
    a = jax.lax.dot_general(magg, Wh[...], (((1,), (0,)), ((), ())),
                            preferred_element_type=jnp.float32)
    h = _expmap0(a, c)
    n = jnp.maximum(jnp.sqrt(jnp.sum(h * h, axis=-1, keepdims=True)), MIN_NORM)
    out1[...] = h / n / sqrt_c * _artanh(sqrt_c * n)
    out2[...] = pp[0] + pp[1]


def _final(cc, Wh, mpart, ppart):
    f32 = jnp.float32
    FB = 80
    return pl.pallas_call(
        _final_body,
        grid=(N_NODE // FB,),
        in_specs=[
            pl.BlockSpec(memory_space=pltpu.SMEM),
            pl.BlockSpec((D, D), lambda i: (0, 0)),
            pl.BlockSpec((NCORE, FB, D), lambda i: (0, i, 0)),
            pl.BlockSpec((NCORE, FB, DP), lambda i: (0, i, 0)),
        ],
        out_specs=[pl.BlockSpec((FB, D), lambda i: (i, 0)),
                   pl.BlockSpec((FB, DP), lambda i: (i, 0))],
        out_shape=[jax.ShapeDtypeStruct((N_NODE, D), f32),
                   jax.ShapeDtypeStruct((N_NODE, DP), f32)],
    )(cc, Wh, mpart, ppart)


def kernel(q_sub, q_rel, hidden, path_state, edges, nodes, old_nodes_new_idx,
           batchsize, rela_embed, Ws_attn, Wr_attn, Wqr_attn_w, Wqr_attn_b,
           w_alpha_w, w_alpha_b, W_h, W_path_prev, W_path_rel, curvature):
    f32 = jnp.float32
    i32 = jnp.int32
    cc = jnp.reshape(jnp.asarray(curvature, f32), (1, 1))
    cvec = jnp.full((16,), jnp.asarray(curvature, f32))
    wb16 = jnp.full((16,), jnp.asarray(w_alpha_b, f32).reshape(())[()])

    hid_p = jnp.zeros((NPAD, D), f32).at[:N_NODE].set(hidden)
    rela_p = jnp.zeros((NPAD, D), f32).at[:rela_embed.shape[0]].set(rela_embed)
    path_p = jnp.zeros((NPAD, DP), f32).at[:N_NODE].set(path_state)
    sub = jnp.asarray(edges[:, 4], i32)
    rel = jnp.asarray(edges[:, 2], i32)
    obj = jnp.asarray(edges[:, 5], i32)
    ridx = jnp.asarray(edges[:, 0], i32)
    qrel = jnp.asarray(q_rel, i32)
    obj3 = jnp.reshape(obj, (NW, NCH, CH))
    obj4 = jnp.reshape(obj, (NW, NCH, 1, CH))
    z128 = jnp.zeros((N_NODE, D), f32)
    z64 = jnp.zeros((N_NODE, DP), f32)

    AS, AR, AQ, PS, PR, HS, HR = _precompute(
        cc, hid_p, rela_p, path_p, Ws_attn, Wr_attn, Wqr_attn_w,
        jnp.reshape(Wqr_attn_b, (1, D)), W_path_prev, W_path_rel)

    msg, ppart = _fused_edge(qrel, ridx, sub, rel, obj4,
                             jnp.reshape(w_alpha_w, (D,)), wb16, cvec, z64,
                             AS, AR, AQ, HS, HR, PS, PR)
    mpart = _scatter(msg, obj3, z128, D)

    out1, out2 = _final(cc, W_h, mpart, ppart)
    return (out1, out2)


# speedup vs baseline: 4.4199x; 4.4199x over previous
import functools

import jax
import jax.numpy as jnp
from jax import lax
from jax.experimental import pallas as pl
from jax.experimental.pallas import tpu as pltpu
from jax.experimental.pallas import tpu_sc as plsc

MIN_NORM = 1e-15
BALL_EPS = 0.004
MIN_CURVATURE = 1e-06

N_NODE = 10000
N_EDGE = 320000
D = 128
DP = 64
NPAD = 10112
NBLK = 79
NCORE = 2
NSUB = 16
NW = NCORE * NSUB
EPW = N_EDGE // NW
CH = 80
NCH = EPW // CH
RPT = N_NODE // NSUB


def _tanh_c(x):
    return jnp.tanh(jnp.clip(x, -15.0, 15.0))


def _artanh(x):
    x = jnp.clip(x, -1.0 + 1e-05, 1.0 - 1e-05)
    return 0.5 * (jnp.log1p(x) - jnp.log1p(-x))


def _expmap0(u, c):
    sqrt_c = jnp.sqrt(c)
    u_norm = jnp.maximum(jnp.sqrt(jnp.sum(u * u, axis=-1, keepdims=True)), MIN_NORM)
    gamma = _tanh_c(sqrt_c * u_norm) * u / (sqrt_c * u_norm)
    norm = jnp.maximum(jnp.sqrt(jnp.sum(gamma * gamma, axis=-1, keepdims=True)), MIN_NORM)
    maxnorm = (1.0 - BALL_EPS) / sqrt_c
    return jnp.where(norm > maxnorm, gamma / norm * maxnorm, gamma)


def _pre_body(c_ref, hid, rela, path, Ws, Wr, Wq, bq, Wpp, Wpr,
              AS, AR, AQ, PS, PR, HS, HR):
    c = jnp.maximum(c_ref[0, 0], MIN_CURVATURE)
    h = hid[...]
    r = rela[...]
    AS[...] = jax.lax.dot_general(h, Ws[...], (((1,), (0,)), ((), ())),
                                  preferred_element_type=jnp.float32)
    AR[...] = jax.lax.dot_general(r, Wr[...], (((1,), (0,)), ((), ())),
                                  preferred_element_type=jnp.float32)
    AQ[...] = jax.lax.dot_general(r, Wq[...], (((1,), (0,)), ((), ())),
                                  preferred_element_type=jnp.float32) + bq[...]
    PS[...] = jax.lax.dot_general(path[...], Wpp[...], (((1,), (0,)), ((), ())),
                                  preferred_element_type=jnp.float32)
    PR[...] = jax.lax.dot_general(r, Wpr[...], (((1,), (0,)), ((), ())),
                                  preferred_element_type=jnp.float32)
    HS[...] = _expmap0(h, c)
    HR[...] = _expmap0(r, c)


def _precompute(cc, hid_p, rela_p, path_p, Ws, Wr, Wq, bq, Wpp, Wpr):
    f32 = jnp.float32
    full = lambda s: pl.BlockSpec(s, lambda i: (0,) * len(s))
    row128 = pl.BlockSpec((D, D), lambda i: (i, 0))
    row64 = pl.BlockSpec((D, DP), lambda i: (i, 0))
    return pl.pallas_call(
        _pre_body,
        grid=(NBLK,),
        in_specs=[
            pl.BlockSpec(memory_space=pltpu.SMEM),
            row128, row128, row64,
            full((D, D)), full((D, D)), full((D, D)), full((1, D)),
            full((DP, DP)), full((D, DP)),
        ],
        out_specs=[row128, row128, row128, row64, row64, row128, row128],
        out_shape=[
            jax.ShapeDtypeStruct((NPAD, D), f32),
            jax.ShapeDtypeStruct((NPAD, D), f32),
            jax.ShapeDtypeStruct((NPAD, D), f32),
            jax.ShapeDtypeStruct((NPAD, DP), f32),
            jax.ShapeDtypeStruct((NPAD, DP), f32),
            jax.ShapeDtypeStruct((NPAD, D), f32),
            jax.ShapeDtypeStruct((NPAD, D), f32),
        ],
    )(cc, hid_p, rela_p, path_p, Ws, Wr, Wq, bq, Wpp, Wpr)


CHG = 40
NCHG = EPW // CHG
NPAIR = NCHG // 2


def _gather_body(qrel_hbm, ridx_hbm, sub_hbm, rel_hbm,
                 AS_hbm, AR_hbm, AQ_hbm, HS_hbm, HR_hbm, PS_hbm, PR_hbm,
                 ATT, HSg, HRg, PP,
                 ridx_v, sub_v, rel_v, qi_v,
                 a_as, a_ar, a_aq, a_hs, a_hr, a_ps, a_pr,
                 c_as, c_ar, c_aq, c_hs, c_hr, c_ps, c_pr,
                 semGA, semGB, semOA, semOB):
    cid = lax.axis_index("c")
    sid = lax.axis_index("s")
    wid = sid * NCORE + cid
    base = wid * EPW
    pltpu.sync_copy(ridx_hbm.at[pl.ds(base, EPW)], ridx_v)
    pltpu.sync_copy(sub_hbm.at[pl.ds(base, EPW)], sub_v)
    pltpu.sync_copy(rel_hbm.at[pl.ds(base, EPW)], rel_v)
    pltpu.async_copy(qrel_hbm.at[ridx_v], qi_v, semGA).wait()

    bufsA = (a_as, a_ar, a_aq, a_hs, a_hr, a_ps, a_pr)
    bufsB = (c_as, c_ar, c_aq, c_hs, c_hr, c_ps, c_pr)

    def srcs(c):
        off = c * CHG
        s = sub_v.at[pl.ds(off, CHG)]
        r = rel_v.at[pl.ds(off, CHG)]
        q = qi_v.at[pl.ds(off, CHG)]
        return (AS_hbm.at[s], AR_hbm.at[r], AQ_hbm.at[q], HS_hbm.at[s],
                HR_hbm.at[r], PS_hbm.at[s], PR_hbm.at[r])

    def issue_g(bufs, sem, c):
        for src, buf in zip(srcs(c), bufs):
            pltpu.async_copy(src, buf, sem)

    def drain_g(bufs, sem, c):
        for src, buf in zip(srcs(c), bufs):
            pltpu.make_async_copy(src, buf, sem).wait()

    def compute(bufs):
        (b_as, b_ar, b_aq, b_hs, b_hr, b_ps, b_pr) = bufs

        def row(e, _):
            for j in range(8):
                sl = pl.ds(j * 16, 16)
                b_as[e, sl] = b_as[e, sl] + b_ar[e, sl] + b_aq[e, sl]
            for j in range(4):
                sl = pl.ds(j * 16, 16)
                b_ps[e, sl] = b_ps[e, sl] + b_pr[e, sl]
            return 0

        lax.fori_loop(0, CHG, row, 0)

    def outrefs(bufs, c):
        off = base + c * CHG
        return ((bufs[0], ATT.at[pl.ds(off, CHG)]),
                (bufs[3], HSg.at[pl.ds(off, CHG)]),
                (bufs[4], HRg.at[pl.ds(off, CHG)]),
                (bufs[5], PP.at[pl.ds(off, CHG)]))

    def issue_o(bufs, sem, c):
        for buf, dst in outrefs(bufs, c):
            pltpu.async_copy(buf, dst, sem)

    def drain_o(bufs, sem, c):
        for buf, dst in outrefs(bufs, c):
            pltpu.make_async_copy(buf, dst, sem).wait()

    issue_g(bufsA, semGA, 0)

    def pair(k2, _):
        c0 = 2 * k2
        c1 = c0 + 1

        @pl.when(k2 > 0)
        def _():
            drain_o(bufsB, semOB, c1 - 2)

        issue_g(bufsB, semGB, c1)
        drain_g(bufsA, semGA, c0)
        compute(bufsA)
        issue_o(bufsA, semOA, c0)
        drain_g(bufsB, semGB, c1)
        compute(bufsB)
        issue_o(bufsB, semOB, c1)

        @pl.when(k2 < NPAIR - 1)
        def _():
            drain_o(bufsA, semOA, c0)
            issue_g(bufsA, semGA, c0 + 2)

        return 0

    lax.fori_loop(0, NPAIR, pair, 0)
    drain_o(bufsA, semOA, NCHG - 2)
    drain_o(bufsB, semOB, NCHG - 1)


def _edge_gather(qrel, ridx, sub, rel, AS, AR, AQ, HS, HR, PS, PR):
    f32 = jnp.float32
    i32 = jnp.int32
    mesh = plsc.VectorSubcoreMesh(core_axis_name="c", subcore_axis_name="s",
                                  num_cores=NCORE, num_subcores=NSUB)
    out_type = (
        jax.ShapeDtypeStruct((N_EDGE, D), f32),
        jax.ShapeDtypeStruct((N_EDGE, D), f32),
        jax.ShapeDtypeStruct((N_EDGE, D), f32),
        jax.ShapeDtypeStruct((N_EDGE, DP), f32),
    )
    dbl = []
    for _ in range(2):
        dbl += [pltpu.VMEM((CHG, D), f32)] * 5 + [pltpu.VMEM((CHG, DP), f32)] * 2
    scratch = [
        pltpu.VMEM((EPW,), i32),
        pltpu.VMEM((EPW,), i32),
        pltpu.VMEM((EPW,), i32),
        pltpu.VMEM((EPW,), i32),
    ] + dbl + [
        pltpu.SemaphoreType.DMA,
        pltpu.SemaphoreType.DMA,
        pltpu.SemaphoreType.DMA,
        pltpu.SemaphoreType.DMA,
    ]
    k = pl.kernel(_gather_body, out_type=out_type, mesh=mesh, scratch_types=scratch,
                  compiler_params=pltpu.CompilerParams(use_tc_tiling_on_sc=False))
    return k(qrel, ridx, sub, rel, AS, AR, AQ, HS, HR, PS, PR)


EB = 4000
NEB = N_EDGE // EB


def _edge_body(c_ref, wa, wb_ref, att_ref, hsg, hrg, ppg, msg, pe):
    c = jnp.maximum(c_ref[0, 0], MIN_CURVATURE)
    sqrt_c = jnp.sqrt(c)
    att = jnp.maximum(att_ref[...], 0.0)
    logit = jax.lax.dot_general(att, wa[...], (((1,), (0,)), ((), ())),
                                preferred_element_type=jnp.float32) + wb_ref[0, 0]
    alpha = jax.nn.sigmoid(logit)
    hs = hsg[...]
    hr = hrg[...]
    x2 = jnp.sum(hs * hs, axis=-1, keepdims=True)
    y2 = jnp.sum(hr * hr, axis=-1, keepdims=True)
    xy = jnp.sum(hs * hr, axis=-1, keepdims=True)
    A = 1.0 + 2.0 * c * xy + c * y2
    B = 1.0 - c * x2
    den = jnp.maximum(1.0 + 2.0 * c * xy + c * c * x2 * y2, MIN_NORM)
    r2 = A * A * x2 + 2.0 * A * B * xy + B * B * y2
    nm0 = jnp.maximum(jnp.sqrt(jnp.maximum(r2, 0.0)) / den, MIN_NORM)
    maxnorm = (1.0 - BALL_EPS) / sqrt_c
    s1 = jnp.where(nm0 > maxnorm, maxnorm / nm0, 1.0) / den
    n2 = jnp.maximum(jnp.minimum(nm0, maxnorm), MIN_NORM)
    factor = _artanh(sqrt_c * n2) / (n2 * sqrt_c)
    p = alpha * factor * s1 * A
    q = alpha * factor * s1 * B
    msg[...] = p * hs + q * hr
    pe[...] = alpha * _tanh_c(ppg[...])


def _edge_math(cc, wa, wb, ATT, HSg, HRg, PP):
    f32 = jnp.float32
    blk128 = pl.BlockSpec((EB, D), lambda i: (i, 0))
    blk64 = pl.BlockSpec((EB, DP), lambda i: (i, 0))
    return pl.pallas_call(
        _edge_body,
        grid=(NEB,),
        in_specs=[
            pl.BlockSpec(memory_space=pltpu.SMEM),
            pl.BlockSpec((D, 1), lambda i: (0, 0)),
            pl.BlockSpec(memory_space=pltpu.SMEM),
            blk128, blk128, blk128, blk64,
        ],
        out_specs=[blk128, blk64],
        out_shape=[jax.ShapeDtypeStruct((N_EDGE, D), f32),
                   jax.ShapeDtypeStruct((N_EDGE, DP), f32)],
    )(cc, wa, wb, ATT, HSg, HRg, PP)


CHS = 40
NCHS = EPW // CHS
NPAIRS = NCHS // 2


def _scatter_body(msg_hbm, obj_hbm, z_hbm, part,
                  obj_v, bufA, bufB, acc, semA, semB):
    cid = lax.axis_index("c")
    sid = lax.axis_index("s")
    wslot = cid * NSUB + sid
    ebase = wslot * EPW
    pltpu.sync_copy(z_hbm.at[pl.ds(sid * RPT, RPT)], acc.at[pl.ds(sid * RPT, RPT)])
    pltpu.sync_copy(obj_hbm.at[wslot], obj_v)
    plsc.subcore_barrier()

    def rd(c, buf, sem):
        pltpu.async_copy(msg_hbm.at[pl.ds(ebase + c * CHS, CHS)], buf, sem)

    def rd_drain(c, buf, sem):
        pltpu.make_async_copy(msg_hbm.at[pl.ds(ebase + c * CHS, CHS)], buf,
                              sem).wait()

    rd(0, bufA, semA)
    rd(1, bufB, semB)

    def pairstep(k2, _):
        c0 = 2 * k2
        c1 = c0 + 1
        rd_drain(c0, bufA, semA)
        pltpu.sync_copy(bufA, acc.at[obj_v.at[c0]], add=True)

        @pl.when(k2 < NPAIRS - 1)
        def _():
            rd(c0 + 2, bufA, semA)

        rd_drain(c1, bufB, semB)
        pltpu.sync_copy(bufB, acc.at[obj_v.at[c1]], add=True)

        @pl.when(k2 < NPAIRS - 1)
        def _():
            rd(c1 + 2, bufB, semB)

        return 0

    lax.fori_loop(0, NPAIRS, pairstep, 0)
    plsc.subcore_barrier()
    pltpu.sync_copy(acc.at[pl.ds(sid * RPT, RPT)],
                    part.at[cid].at[pl.ds(sid * RPT, RPT)])


def _scatter(msg, obj3, z, width):
    f32 = jnp.float32
    i32 = jnp.int32
    mesh = plsc.VectorSubcoreMesh(core_axis_name="c", subcore_axis_name="s",
                                  num_cores=NCORE, num_subcores=NSUB)
    out_type = jax.ShapeDtypeStruct((NCORE, N_NODE, width), f32)
    scratch = [
        pltpu.VMEM((NCHS, CHS), i32),
        pltpu.VMEM((CHS, width), f32),
        pltpu.VMEM((CHS, width), f32),
        pltpu.VMEM_SHARED((N_NODE, width), f32),
        pltpu.SemaphoreType.DMA,
        pltpu.SemaphoreType.DMA,
    ]
    k = pl.kernel(_scatter_body, out_type=out_type, mesh=mesh, scratch_types=scratch,
                  compiler_params=pltpu.CompilerParams(use_tc_tiling_on_sc=False))
    return k(msg, obj3, z)


def _final_body(c_ref, Wh, mp, pp, out1, out2):
    c = jnp.maximum(c_ref[0, 0], MIN_CURVATURE)
    sqrt_c = jnp.sqrt(c)
    magg = mp[0] + mp[1]
    a = jax.lax.dot_general(magg, Wh[...], (((1,), (0,)), ((), ())),
                            preferred_element_type=jnp.float32)
    h = _expmap0(a, c)
    n = jnp.maximum(jnp.sqrt(jnp.sum(h * h, axis=-1, keepdims=True)), MIN_NORM)
    out1[...] = h / n / sqrt_c * _artanh(sqrt_c * n)
    out2[...] = pp[0] + pp[1]


def _final(cc, Wh, mpart, ppart):
    f32 = jnp.float32
    FB = 80
    return pl.pallas_call(
        _final_body,
        grid=(N_NODE // FB,),
        in_specs=[
            pl.BlockSpec(memory_space=pltpu.SMEM),
            pl.BlockSpec((D, D), lambda i: (0, 0)),
            pl.BlockSpec((NCORE, FB, D), lambda i: (0, i, 0)),
            pl.BlockSpec((NCORE, FB, DP), lambda i: (0, i, 0)),
        ],
        out_specs=[pl.BlockSpec((FB, D), lambda i: (i, 0)),
                   pl.BlockSpec((FB, DP), lambda i: (i, 0))],
        out_shape=[jax.ShapeDtypeStruct((N_NODE, D), f32),
                   jax.ShapeDtypeStruct((N_NODE, DP), f32)],
    )(cc, Wh, mpart, ppart)


def kernel(q_sub, q_rel, hidden, path_state, edges, nodes, old_nodes_new_idx,
           batchsize, rela_embed, Ws_attn, Wr_attn, Wqr_attn_w, Wqr_attn_b,
           w_alpha_w, w_alpha_b, W_h, W_path_prev, W_path_rel, curvature):
    f32 = jnp.float32
    i32 = jnp.int32
    cc = jnp.reshape(jnp.asarray(curvature, f32), (1, 1))

    hid_p = jnp.zeros((NPAD, D), f32).at[:N_NODE].set(hidden)
    rela_p = jnp.zeros((NPAD, D), f32).at[:rela_embed.shape[0]].set(rela_embed)
    path_p = jnp.zeros((NPAD, DP), f32).at[:N_NODE].set(path_state)
    sub = jnp.asarray(edges[:, 4], i32)
    rel = jnp.asarray(edges[:, 2], i32)
    obj = jnp.asarray(edges[:, 5], i32)
    ridx = jnp.asarray(edges[:, 0], i32)
    qrel = jnp.asarray(q_rel, i32)
    obj3 = jnp.reshape(obj, (NW, NCHS, CHS))
    z128 = jnp.zeros((N_NODE, D), f32)
    z64 = jnp.zeros((N_NODE, DP), f32)

    AS, AR, AQ, PS, PR, HS, HR = _precompute(
        cc, hid_p, rela_p, path_p, Ws_attn, Wr_attn, Wqr_attn_w,
        jnp.reshape(Wqr_attn_b, (1, D)), W_path_prev, W_path_rel)

    ATT, HSg, HRg, PP = _edge_gather(qrel, ridx, sub, rel,
                                     AS, AR, AQ, HS, HR, PS, PR)
    wb = jnp.reshape(jnp.asarray(w_alpha_b, f32), (1, 1))
    msg, pe = _edge_math(cc, jnp.reshape(w_alpha_w, (D, 1)), wb,
                         ATT, HSg, HRg, PP)
    mpart = _scatter(msg, obj3, z128, D)
    ppart = _scatter(pe, obj3, z64, DP)

    out1, out2 = _final(cc, W_h, mpart, ppart)
    return (out1, out2)
